# Initial kernel scaffold; baseline (speedup 1.0000x reference)
import functools

import jax
import jax.numpy as jnp
from jax import lax
from jax.experimental import pallas as pl

_BIG = jnp.float32(1e30)
_EPS_D = jnp.float32(1e-10)
_EPS_BN = jnp.float32(1e-5)


def _p1_body(xyz1_ref, xyz2_ref, f1_ref, f2_ref, w1_ref, b1_ref,
             h1_ref, st_ref):
    b = pl.program_id(0)
    i = pl.program_id(1)

    q = xyz1_ref[0]
    k = xyz2_ref[0]
    c2 = f2_ref.shape[1]

    ones_r = jnp.ones((1, 3), jnp.float32)
    ones_c = jnp.ones((3, 1), jnp.float32)
    qn = lax.dot_general(ones_r, q * q, (((1,), (0,)), ((), ())),
                         preferred_element_type=jnp.float32)
    kn = lax.dot_general(k * k, ones_c, (((0,), (0,)), ((), ())),
                         preferred_element_type=jnp.float32)
    cross = lax.dot_general(k, q, (((0,), (0,)), ((), ())),
                            preferred_element_type=jnp.float32)
    sq = jnp.maximum(kn + qn - 2.0 * cross, 0.0)

    d = sq
    m3 = None
    for it in range(3):
        m3 = jnp.min(d, axis=0, keepdims=True)
        if it < 2:
            d = jnp.where(d == m3, _BIG, d)

    w = jnp.where(sq <= m3, 1.0 / jnp.maximum(sq, _EPS_D), 0.0)
    rws = 1.0 / jnp.sum(w, axis=0, keepdims=True)
    s = w * rws

    interp = lax.dot_general(f2_ref[0], s, (((1,), (0,)), ((), ())),
                             preferred_element_type=jnp.float32)

    h1 = (lax.dot_general(w1_ref[:, :c2], interp, (((1,), (0,)), ((), ())),
                          preferred_element_type=jnp.float32)
          + lax.dot_general(w1_ref[:, c2:], f1_ref[0],
                            (((1,), (0,)), ((), ())),
                            preferred_element_type=jnp.float32)
          + b1_ref[...])
    h1_ref[0] = h1

    st = jnp.concatenate(
        [jnp.sum(h1, axis=1, keepdims=True),
         jnp.sum(h1 * h1, axis=1, keepdims=True)], axis=1)

    @pl.when(jnp.logical_and(b == 0, i == 0))
    def _():
        st_ref[...] = jnp.zeros_like(st_ref)

    st_ref[...] = st_ref[...] + st


def _p2_body(minv, h1_ref, st1_ref, g1_ref, be1_ref, w2_ref, b2_ref,
             h2_ref, st_ref):
    b = pl.program_id(0)
    mean = st1_ref[:, 0:1] * minv
    var = st1_ref[:, 1:2] * minv - mean * mean
    scale = g1_ref[...] / jnp.sqrt(var + _EPS_BN)
    shift = be1_ref[...] - mean * scale

    y = jnp.maximum(h1_ref[0] * scale + shift, 0.0)
    h2 = lax.dot_general(w2_ref[...], y, (((1,), (0,)), ((), ())),
                         preferred_element_type=jnp.float32) + b2_ref[...]
    h2_ref[0] = h2

    st = jnp.concatenate(
        [jnp.sum(h2, axis=1, keepdims=True),
         jnp.sum(h2 * h2, axis=1, keepdims=True)], axis=1)

    @pl.when(b == 0)
    def _():
        st_ref[...] = jnp.zeros_like(st_ref)

    st_ref[...] = st_ref[...] + st


def _p3_body(minv, h2_ref, st2_ref, g2_ref, be2_ref, out_ref):
    mean = st2_ref[:, 0:1] * minv
    var = st2_ref[:, 1:2] * minv - mean * mean
    scale = g2_ref[...] / jnp.sqrt(var + _EPS_BN)
    shift = be2_ref[...] - mean * scale
    out_ref[0] = jnp.maximum(h2_ref[0] * scale + shift, 0.0)


def kernel(xyz1, xyz2, features1, features2, W1, b1, g1, be1, W2, b2, g2,
           be2):
    B, _, N1 = xyz1.shape
    N2 = xyz2.shape[2]
    C1 = features1.shape[1]
    C2 = features2.shape[1]
    Co = W1.shape[0]
    T = 256
    NB = N1 // T
    minv = 1.0 / float(B * N1)

    h1, st1 = pl.pallas_call(
        _p1_body,
        grid=(B, NB),
        in_specs=[
            pl.BlockSpec((1, 3, T), lambda b, i: (b, 0, i)),
            pl.BlockSpec((1, 3, N2), lambda b, i: (b, 0, 0)),
            pl.BlockSpec((1, C1, T), lambda b, i: (b, 0, i)),
            pl.BlockSpec((1, C2, N2), lambda b, i: (b, 0, 0)),
            pl.BlockSpec((Co, C1 + C2), lambda b, i: (0, 0)),
            pl.BlockSpec((Co, 1), lambda b, i: (0, 0)),
        ],
        out_specs=[
            pl.BlockSpec((1, Co, T), lambda b, i: (b, 0, i)),
            pl.BlockSpec((Co, 2), lambda b, i: (0, 0)),
        ],
        out_shape=[
            jax.ShapeDtypeStruct((B, Co, N1), jnp.float32),
            jax.ShapeDtypeStruct((Co, 2), jnp.float32),
        ],
    )(xyz1, xyz2, features1, features2, W1, b1.reshape(Co, 1))

    h2, st2 = pl.pallas_call(
        functools.partial(_p2_body, minv),
        grid=(B,),
        in_specs=[
            pl.BlockSpec((1, Co, N1), lambda b: (b, 0, 0)),
            pl.BlockSpec((Co, 2), lambda b: (0, 0)),
            pl.BlockSpec((Co, 1), lambda b: (0, 0)),
            pl.BlockSpec((Co, 1), lambda b: (0, 0)),
            pl.BlockSpec((Co, Co), lambda b: (0, 0)),
            pl.BlockSpec((Co, 1), lambda b: (0, 0)),
        ],
        out_specs=[
            pl.BlockSpec((1, Co, N1), lambda b: (b, 0, 0)),
            pl.BlockSpec((Co, 2), lambda b: (0, 0)),
        ],
        out_shape=[
            jax.ShapeDtypeStruct((B, Co, N1), jnp.float32),
            jax.ShapeDtypeStruct((Co, 2), jnp.float32),
        ],
    )(h1, st1, g1.reshape(Co, 1), be1.reshape(Co, 1), W2,
      b2.reshape(Co, 1))

    out = pl.pallas_call(
        functools.partial(_p3_body, minv),
        grid=(B,),
        in_specs=[
            pl.BlockSpec((1, Co, N1), lambda b: (b, 0, 0)),
            pl.BlockSpec((Co, 2), lambda b: (0, 0)),
            pl.BlockSpec((Co, 1), lambda b: (0, 0)),
            pl.BlockSpec((Co, 1), lambda b: (0, 0)),
        ],
        out_specs=pl.BlockSpec((1, Co, N1), lambda b: (b, 0, 0)),
        out_shape=jax.ShapeDtypeStruct((B, Co, N1), jnp.float32),
    )(h2, st2, g2.reshape(Co, 1), be2.reshape(Co, 1))

    return out


# speedup vs baseline: 24.3677x; 24.3677x over previous
import functools

import jax
import jax.numpy as jnp
from jax import lax
from jax.experimental import pallas as pl

_BIG = 1e30
_EPS_D = 1e-10
_EPS_BN = 1e-5


def _p1_body(xyz1_ref, xyz2t_ref, f1_ref, f2_ref, w1_ref, b1_ref,
             h1_ref, st_ref):
    b = pl.program_id(0)
    i = pl.program_id(1)

    q = xyz1_ref[0]
    kc = xyz2t_ref[0]
    c2 = f2_ref.shape[1]

    qn = jnp.sum(q * q, axis=0, keepdims=True)
    kn = jnp.sum(kc * kc, axis=1, keepdims=True)
    cross = lax.dot_general(kc, q, (((1,), (0,)), ((), ())),
                            preferred_element_type=jnp.float32)
    sq = jnp.maximum(kn + qn - 2.0 * cross, 0.0)

    d = sq
    m3 = None
    for it in range(3):
        m3 = jnp.min(d, axis=0, keepdims=True)
        if it < 2:
            d = jnp.where(d == m3, _BIG, d)

    w = jnp.where(sq <= m3, 1.0 / jnp.maximum(sq, _EPS_D), 0.0)
    rws = 1.0 / jnp.sum(w, axis=0, keepdims=True)

    interp = lax.dot_general(f2_ref[0], w, (((1,), (0,)), ((), ())),
                             preferred_element_type=jnp.float32,
                             precision=lax.Precision.HIGHEST) * rws

    h1 = (lax.dot_general(w1_ref[:, :c2], interp, (((1,), (0,)), ((), ())),
                          preferred_element_type=jnp.float32)
          + lax.dot_general(w1_ref[:, c2:], f1_ref[0],
                            (((1,), (0,)), ((), ())),
                            preferred_element_type=jnp.float32)
          + b1_ref[...])
    h1_ref[0] = h1

    st = jnp.concatenate(
        [jnp.sum(h1, axis=1, keepdims=True),
         jnp.sum(h1 * h1, axis=1, keepdims=True)], axis=1)

    @pl.when(jnp.logical_and(b == 0, i == 0))
    def _():
        st_ref[...] = jnp.zeros_like(st_ref)

    st_ref[...] = st_ref[...] + st


def _p2_body(minv, h1_ref, st1_ref, g1_ref, be1_ref, w2_ref, b2_ref,
             h2_ref, st_ref):
    b = pl.program_id(0)
    mean = st1_ref[:, 0:1] * minv
    var = st1_ref[:, 1:2] * minv - mean * mean
    scale = g1_ref[...] / jnp.sqrt(var + _EPS_BN)
    shift = be1_ref[...] - mean * scale

    y = jnp.maximum(h1_ref[0] * scale + shift, 0.0)
    h2 = lax.dot_general(w2_ref[...], y, (((1,), (0,)), ((), ())),
                         preferred_element_type=jnp.float32) + b2_ref[...]
    h2_ref[0] = h2

    st = jnp.concatenate(
        [jnp.sum(h2, axis=1, keepdims=True),
         jnp.sum(h2 * h2, axis=1, keepdims=True)], axis=1)

    @pl.when(b == 0)
    def _():
        st_ref[...] = jnp.zeros_like(st_ref)

    st_ref[...] = st_ref[...] + st


def _p3_body(minv, h2_ref, st2_ref, g2_ref, be2_ref, out_ref):
    mean = st2_ref[:, 0:1] * minv
    var = st2_ref[:, 1:2] * minv - mean * mean
    scale = g2_ref[...] / jnp.sqrt(var + _EPS_BN)
    shift = be2_ref[...] - mean * scale
    out_ref[0] = jnp.maximum(h2_ref[0] * scale + shift, 0.0)


def kernel(xyz1, xyz2, features1, features2, W1, b1, g1, be1, W2, b2, g2,
           be2):
    B, _, N1 = xyz1.shape
    N2 = xyz2.shape[2]
    C1 = features1.shape[1]
    C2 = features2.shape[1]
    Co = W1.shape[0]
    T = 256
    NB = N1 // T
    minv = 1.0 / float(B * N1)

    h1, st1 = pl.pallas_call(
        _p1_body,
        grid=(B, NB),
        in_specs=[
            pl.BlockSpec((1, 3, T), lambda b, i: (b, 0, i)),
            pl.BlockSpec((1, N2, 3), lambda b, i: (b, 0, 0)),
            pl.BlockSpec((1, C1, T), lambda b, i: (b, 0, i)),
            pl.BlockSpec((1, C2, N2), lambda b, i: (b, 0, 0)),
            pl.BlockSpec((Co, C1 + C2), lambda b, i: (0, 0)),
            pl.BlockSpec((Co, 1), lambda b, i: (0, 0)),
        ],
        out_specs=[
            pl.BlockSpec((1, Co, T), lambda b, i: (b, 0, i)),
            pl.BlockSpec((Co, 2), lambda b, i: (0, 0)),
        ],
        out_shape=[
            jax.ShapeDtypeStruct((B, Co, N1), jnp.float32),
            jax.ShapeDtypeStruct((Co, 2), jnp.float32),
        ],
    )(xyz1, jnp.transpose(xyz2, (0, 2, 1)), features1, features2, W1,
      b1.reshape(Co, 1))

    h2, st2 = pl.pallas_call(
        functools.partial(_p2_body, minv),
        grid=(B,),
        in_specs=[
            pl.BlockSpec((1, Co, N1), lambda b: (b, 0, 0)),
            pl.BlockSpec((Co, 2), lambda b: (0, 0)),
            pl.BlockSpec((Co, 1), lambda b: (0, 0)),
            pl.BlockSpec((Co, 1), lambda b: (0, 0)),
            pl.BlockSpec((Co, Co), lambda b: (0, 0)),
            pl.BlockSpec((Co, 1), lambda b: (0, 0)),
        ],
        out_specs=[
            pl.BlockSpec((1, Co, N1), lambda b: (b, 0, 0)),
            pl.BlockSpec((Co, 2), lambda b: (0, 0)),
        ],
        out_shape=[
            jax.ShapeDtypeStruct((B, Co, N1), jnp.float32),
            jax.ShapeDtypeStruct((Co, 2), jnp.float32),
        ],
    )(h1, st1, g1.reshape(Co, 1), be1.reshape(Co, 1), W2,
      b2.reshape(Co, 1))

    out = pl.pallas_call(
        functools.partial(_p3_body, minv),
        grid=(B,),
        in_specs=[
            pl.BlockSpec((1, Co, N1), lambda b: (b, 0, 0)),
            pl.BlockSpec((Co, 2), lambda b: (0, 0)),
            pl.BlockSpec((Co, 1), lambda b: (0, 0)),
            pl.BlockSpec((Co, 1), lambda b: (0, 0)),
        ],
        out_specs=pl.BlockSpec((1, Co, N1), lambda b: (b, 0, 0)),
        out_shape=jax.ShapeDtypeStruct((B, Co, N1), jnp.float32),
    )(h2, st2, g2.reshape(Co, 1), be2.reshape(Co, 1))

    return out


# speedup vs baseline: 43.1599x; 1.7712x over previous
import functools

import jax
import jax.numpy as jnp
from jax import lax
from jax.experimental import pallas as pl
from jax.experimental.pallas import tpu as pltpu

_BIG = 1e30
_EPS_D = 1e-10
_EPS_BN = 1e-5


def _p1_body(xyz1_ref, xyz2t_ref, f1_ref, f2_ref, w1_ref, b1_ref,
             h1_ref, st_ref, kn_ref):
    b = pl.program_id(0)
    i = pl.program_id(1)

    q = xyz1_ref[0]
    kc = xyz2t_ref[0]
    c2 = f2_ref.shape[1]

    @pl.when(i == 0)
    def _():
        kn_ref[...] = jnp.sum(kc * kc, axis=1, keepdims=True)

    qn = jnp.sum(q * q, axis=0, keepdims=True)
    kn = kn_ref[...]
    cross = lax.dot_general(kc, q, (((1,), (0,)), ((), ())),
                            preferred_element_type=jnp.float32)
    sq = jnp.maximum(kn + qn - 2.0 * cross, 0.0)

    d = sq
    m3 = None
    for it in range(3):
        m3 = jnp.min(d, axis=0, keepdims=True)
        if it < 2:
            d = jnp.where(d == m3, _BIG, d)

    w = jnp.where(sq <= m3, 1.0 / jnp.maximum(sq, _EPS_D), 0.0)
    rws = 1.0 / jnp.sum(w, axis=0, keepdims=True)

    interp = lax.dot_general(f2_ref[0], w, (((1,), (0,)), ((), ())),
                             preferred_element_type=jnp.float32,
                             ) * rws

    h1 = (lax.dot_general(w1_ref[:, :c2], interp, (((1,), (0,)), ((), ())),
                          preferred_element_type=jnp.float32)
          + lax.dot_general(w1_ref[:, c2:], f1_ref[0],
                            (((1,), (0,)), ((), ())),
                            preferred_element_type=jnp.float32)
          + b1_ref[...])
    h1_ref[0] = h1

    st = jnp.concatenate(
        [jnp.sum(h1, axis=1, keepdims=True),
         jnp.sum(h1 * h1, axis=1, keepdims=True)], axis=1)

    @pl.when(jnp.logical_and(b == 0, i == 0))
    def _():
        st_ref[...] = jnp.zeros_like(st_ref)

    st_ref[...] = st_ref[...] + st


def _p2_body(minv, h1_ref, st1_ref, g1_ref, be1_ref, w2_ref, b2_ref,
             h2_ref, st_ref):
    b = pl.program_id(0)
    mean = st1_ref[:, 0:1] * minv
    var = st1_ref[:, 1:2] * minv - mean * mean
    scale = g1_ref[...] / jnp.sqrt(var + _EPS_BN)
    shift = be1_ref[...] - mean * scale

    y = jnp.maximum(h1_ref[0] * scale + shift, 0.0)
    h2 = lax.dot_general(w2_ref[...], y, (((1,), (0,)), ((), ())),
                         preferred_element_type=jnp.float32) + b2_ref[...]
    h2_ref[0] = h2

    st = jnp.concatenate(
        [jnp.sum(h2, axis=1, keepdims=True),
         jnp.sum(h2 * h2, axis=1, keepdims=True)], axis=1)

    @pl.when(b == 0)
    def _():
        st_ref[...] = jnp.zeros_like(st_ref)

    st_ref[...] = st_ref[...] + st


def _p3_body(minv, h2_ref, st2_ref, g2_ref, be2_ref, out_ref):
    mean = st2_ref[:, 0:1] * minv
    var = st2_ref[:, 1:2] * minv - mean * mean
    scale = g2_ref[...] / jnp.sqrt(var + _EPS_BN)
    shift = be2_ref[...] - mean * scale
    out_ref[0] = jnp.maximum(h2_ref[0] * scale + shift, 0.0)


def kernel(xyz1, xyz2, features1, features2, W1, b1, g1, be1, W2, b2, g2,
           be2):
    B, _, N1 = xyz1.shape
    N2 = xyz2.shape[2]
    C1 = features1.shape[1]
    C2 = features2.shape[1]
    Co = W1.shape[0]
    T = 512
    NB = N1 // T
    minv = 1.0 / float(B * N1)

    h1, st1 = pl.pallas_call(
        _p1_body,
        grid=(B, NB),
        in_specs=[
            pl.BlockSpec((1, 3, T), lambda b, i: (b, 0, i)),
            pl.BlockSpec((1, N2, 3), lambda b, i: (b, 0, 0)),
            pl.BlockSpec((1, C1, T), lambda b, i: (b, 0, i)),
            pl.BlockSpec((1, C2, N2), lambda b, i: (b, 0, 0)),
            pl.BlockSpec((Co, C1 + C2), lambda b, i: (0, 0)),
            pl.BlockSpec((Co, 1), lambda b, i: (0, 0)),
        ],
        out_specs=[
            pl.BlockSpec((1, Co, T), lambda b, i: (b, 0, i)),
            pl.BlockSpec((Co, 2), lambda b, i: (0, 0)),
        ],
        out_shape=[
            jax.ShapeDtypeStruct((B, Co, N1), jnp.float32),
            jax.ShapeDtypeStruct((Co, 2), jnp.float32),
        ],
        scratch_shapes=[pltpu.VMEM((N2, 1), jnp.float32)],
    )(xyz1, jnp.transpose(xyz2, (0, 2, 1)), features1, features2, W1,
      b1.reshape(Co, 1))

    h2, st2 = pl.pallas_call(
        functools.partial(_p2_body, minv),
        grid=(B,),
        in_specs=[
            pl.BlockSpec((1, Co, N1), lambda b: (b, 0, 0)),
            pl.BlockSpec((Co, 2), lambda b: (0, 0)),
            pl.BlockSpec((Co, 1), lambda b: (0, 0)),
            pl.BlockSpec((Co, 1), lambda b: (0, 0)),
            pl.BlockSpec((Co, Co), lambda b: (0, 0)),
            pl.BlockSpec((Co, 1), lambda b: (0, 0)),
        ],
        out_specs=[
            pl.BlockSpec((1, Co, N1), lambda b: (b, 0, 0)),
            pl.BlockSpec((Co, 2), lambda b: (0, 0)),
        ],
        out_shape=[
            jax.ShapeDtypeStruct((B, Co, N1), jnp.float32),
            jax.ShapeDtypeStruct((Co, 2), jnp.float32),
        ],
    )(h1, st1, g1.reshape(Co, 1), be1.reshape(Co, 1), W2,
      b2.reshape(Co, 1))

    out = pl.pallas_call(
        functools.partial(_p3_body, minv),
        grid=(B,),
        in_specs=[
            pl.BlockSpec((1, Co, N1), lambda b: (b, 0, 0)),
            pl.BlockSpec((Co, 2), lambda b: (0, 0)),
            pl.BlockSpec((Co, 1), lambda b: (0, 0)),
            pl.BlockSpec((Co, 1), lambda b: (0, 0)),
        ],
        out_specs=pl.BlockSpec((1, Co, N1), lambda b: (b, 0, 0)),
        out_shape=jax.ShapeDtypeStruct((B, Co, N1), jnp.float32),
    )(h2, st2, g2.reshape(Co, 1), be2.reshape(Co, 1))

    return out


# speedup vs baseline: 52.5922x; 1.2185x over previous
import functools

import jax
import jax.numpy as jnp
from jax import lax
from jax.experimental import pallas as pl
from jax.experimental.pallas import tpu as pltpu

_BIG = 1e30
_EPS_D = 1e-10
_EPS_BN = 1e-5


def _p1_body(xyz1_ref, xyz2t_ref, f1_ref, f2_ref, w1_ref, b1_ref,
             h1_ref, st_ref, kn_ref):
    b = pl.program_id(0)
    i = pl.program_id(1)

    q = xyz1_ref[0]
    kc = xyz2t_ref[0]
    c2 = f2_ref.shape[1]

    @pl.when(i == 0)
    def _():
        kn_ref[...] = jnp.sum(kc * kc, axis=1, keepdims=True)

    qn = jnp.sum(q * q, axis=0, keepdims=True)
    kn = kn_ref[...]
    cross = lax.dot_general(kc, q, (((1,), (0,)), ((), ())),
                            preferred_element_type=jnp.float32)
    sq = jnp.maximum(kn + qn - 2.0 * cross, 0.0)

    d = sq
    ms = []
    for it in range(3):
        ms.append(jnp.min(d, axis=0, keepdims=True))
        if it < 2:
            d = jnp.where(d == ms[-1], _BIG, d)
    m3 = ms[2]

    w = jnp.where(sq <= m3, 1.0 / jnp.maximum(sq, _EPS_D), 0.0)
    rsum = (1.0 / jnp.maximum(ms[0], _EPS_D)
            + 1.0 / jnp.maximum(ms[1], _EPS_D)
            + 1.0 / jnp.maximum(ms[2], _EPS_D))
    rws = 1.0 / rsum

    interp = lax.dot_general(f2_ref[0], w, (((1,), (0,)), ((), ())),
                             preferred_element_type=jnp.float32,
                             ) * rws

    h1 = (lax.dot_general(w1_ref[:, :c2], interp, (((1,), (0,)), ((), ())),
                          preferred_element_type=jnp.float32)
          + lax.dot_general(w1_ref[:, c2:], f1_ref[0],
                            (((1,), (0,)), ((), ())),
                            preferred_element_type=jnp.float32)
          + b1_ref[...])
    h1_ref[0] = h1

    st = jnp.concatenate(
        [jnp.sum(h1, axis=1, keepdims=True),
         jnp.sum(h1 * h1, axis=1, keepdims=True)], axis=1)

    @pl.when(jnp.logical_and(b == 0, i == 0))
    def _():
        st_ref[...] = jnp.zeros_like(st_ref)

    st_ref[...] = st_ref[...] + st


def _p23_body(minv, h1_ref, st1_ref, g1_ref, be1_ref, w2_ref, b2_ref,
              g2_ref, be2_ref, out_ref, h2_vmem, st2_vmem):
    p = pl.program_id(0)
    b = pl.program_id(1)
    nb = pl.num_programs(1)

    @pl.when(p == 0)
    def _():
        mean = st1_ref[:, 0:1] * minv
        var = st1_ref[:, 1:2] * minv - mean * mean
        scale = g1_ref[...] / jnp.sqrt(var + _EPS_BN)
        shift = be1_ref[...] - mean * scale

        y = jnp.maximum(h1_ref[0] * scale + shift, 0.0)
        h2 = lax.dot_general(w2_ref[...], y, (((1,), (0,)), ((), ())),
                             preferred_element_type=jnp.float32) + b2_ref[...]
        h2_vmem[b] = h2

        st = jnp.concatenate(
            [jnp.sum(h2, axis=1, keepdims=True),
             jnp.sum(h2 * h2, axis=1, keepdims=True)], axis=1)
        prev = jnp.where(b == 0, jnp.zeros_like(st), st2_vmem[...])
        st2_vmem[...] = prev + st

    @pl.when(p == 1)
    def _():
        mean = st2_vmem[:, 0:1] * minv
        var = st2_vmem[:, 1:2] * minv - mean * mean
        scale = g2_ref[...] / jnp.sqrt(var + _EPS_BN)
        shift = be2_ref[...] - mean * scale
        out_ref[0] = jnp.maximum(h2_vmem[b] * scale + shift, 0.0)


def kernel(xyz1, xyz2, features1, features2, W1, b1, g1, be1, W2, b2, g2,
           be2):
    B, _, N1 = xyz1.shape
    N2 = xyz2.shape[2]
    C1 = features1.shape[1]
    C2 = features2.shape[1]
    Co = W1.shape[0]
    T = 1024
    NB = N1 // T
    minv = 1.0 / float(B * N1)

    h1, st1 = pl.pallas_call(
        _p1_body,
        grid=(B, NB),
        in_specs=[
            pl.BlockSpec((1, 3, T), lambda b, i: (b, 0, i)),
            pl.BlockSpec((1, N2, 3), lambda b, i: (b, 0, 0)),
            pl.BlockSpec((1, C1, T), lambda b, i: (b, 0, i)),
            pl.BlockSpec((1, C2, N2), lambda b, i: (b, 0, 0)),
            pl.BlockSpec((Co, C1 + C2), lambda b, i: (0, 0)),
            pl.BlockSpec((Co, 1), lambda b, i: (0, 0)),
        ],
        out_specs=[
            pl.BlockSpec((1, Co, T), lambda b, i: (b, 0, i)),
            pl.BlockSpec((Co, 2), lambda b, i: (0, 0)),
        ],
        out_shape=[
            jax.ShapeDtypeStruct((B, Co, N1), jnp.float32),
            jax.ShapeDtypeStruct((Co, 2), jnp.float32),
        ],
        scratch_shapes=[pltpu.VMEM((N2, 1), jnp.float32)],
    )(xyz1, jnp.transpose(xyz2, (0, 2, 1)), features1, features2, W1,
      b1.reshape(Co, 1))

    out = pl.pallas_call(
        functools.partial(_p23_body, minv),
        grid=(2, B),
        in_specs=[
            pl.BlockSpec((1, Co, N1),
                         lambda p, b: (jnp.where(p == 0, b, B - 1), 0, 0)),
            pl.BlockSpec((Co, 2), lambda p, b: (0, 0)),
            pl.BlockSpec((Co, 1), lambda p, b: (0, 0)),
            pl.BlockSpec((Co, 1), lambda p, b: (0, 0)),
            pl.BlockSpec((Co, Co), lambda p, b: (0, 0)),
            pl.BlockSpec((Co, 1), lambda p, b: (0, 0)),
            pl.BlockSpec((Co, 1), lambda p, b: (0, 0)),
            pl.BlockSpec((Co, 1), lambda p, b: (0, 0)),
        ],
        out_specs=pl.BlockSpec(
            (1, Co, N1), lambda p, b: (jnp.where(p == 1, b, 0), 0, 0)),
        out_shape=jax.ShapeDtypeStruct((B, Co, N1), jnp.float32),
        scratch_shapes=[
            pltpu.VMEM((B, Co, N1), jnp.float32),
            pltpu.VMEM((Co, 2), jnp.float32),
        ],
    )(h1, st1, g1.reshape(Co, 1), be1.reshape(Co, 1), W2,
      b2.reshape(Co, 1), g2.reshape(Co, 1), be2.reshape(Co, 1))

    return out
